# baseline (speedup 1.0000x reference)
import functools

import jax
import jax.numpy as jnp
from jax import lax
from jax.experimental import pallas as pl
from jax.experimental.pallas import tpu as pltpu
from jax.experimental.pallas import tpu_sc as plsc

N = 32768
H = 768
L = 16
HV = H // L
NC, NS = 2, 16
NW = NC * NS
NPW = N // NW
CT = 4
G = 10
GR = G * CT
NCH = NPW // CT
NQ = NCH // 4
INV_H = 1.0 / H
EPS = 1e-12

_SIZES = (512, 2, 128, 1024, 1024, 1024, 1024)
_OFF = []
_acc = 0
for _s in _SIZES:
    _OFF.append(_acc)
    _acc += _s
R_SMALL = _acc
GS = 48


def _rsqrt(x):
    xi = lax.bitcast_convert_type(x, jnp.int32)
    y = lax.bitcast_convert_type(0x5F3759DF - (xi >> 1), jnp.float32)
    for _ in range(3):
        y = y * (1.5 - 0.5 * x * y * y)
    return y


def _reduce_splat(v):
    dnums = lax.GatherDimensionNumbers(
        offset_dims=(), collapsed_slice_dims=(0,), start_index_map=(0,))
    for off in (8, 4, 2, 1):
        perm = lax.iota(jnp.int32, L) ^ off
        v = v + lax.gather(v, perm[:, None], dnums, (1,),
                           mode=lax.GatherScatterMode.PROMISE_IN_BOUNDS)
    return v


def _body(word_hbm, small_hbm, idx_hbm, gb_hbm, out_hbm,
          idx_v, bw_a, bs_a, bw_b, bs_b, obuf, gb_v, s_a, s_b, s_o0, s_o1):
    wid = lax.axis_index("s") * NC + lax.axis_index("c")
    base = wid * NPW
    pltpu.sync_copy(gb_hbm, gb_v)
    pltpu.sync_copy(idx_hbm.at[pl.ds(wid * (NCH * GS), NCH * GS)], idx_v)

    def _descs(c, bw, bs, sem):
        word = pltpu.make_async_copy(
            word_hbm.at[idx_v.at[pl.ds(c * GS, 8)]], bw, sem)
        small = pltpu.make_async_copy(
            small_hbm.at[idx_v.at[pl.ds(c * GS + 8, 40)]], bs, sem)
        return word, small

    def gather(c, bw, bs, sem):
        for d in _descs(c, bw, bs, sem):
            d.start()

    def gather_wait(c, bw, bs, sem):
        for d in _descs(c, bw, bs, sem):
            d.wait()

    def out_desc(p, slot, sem):
        return pltpu.make_async_copy(
            obuf.at[slot], out_hbm.at[pl.ds(base + p * 8, 8)], sem)

    U = 4

    def token(bw, bs, t, slot, j):
        refs = [(bw, t)] + [(bs, k * CT + t) for k in range(G - 1)]

        def p1(ii, carry):
            s, q = carry
            for u in range(U):
                sl = pl.ds((ii * U + u) * L, L)
                vs = [ref[r, sl] for ref, r in refs]
                while len(vs) > 1:
                    vs = [a + b for a, b in zip(vs[::2], vs[1::2])] + (
                        [vs[-1]] if len(vs) % 2 else [])
                v = vs[0]
                obuf[slot, j, sl] = v
                s = s + v
                q = q + v * v
            return s, q
        z = jnp.zeros((L,), jnp.float32)
        s, q = lax.fori_loop(0, HV // U, p1, (z, z))
        mu = _reduce_splat(s) * INV_H
        var = _reduce_splat(q) * INV_H - mu * mu
        rstd = _rsqrt(var + EPS)

        def p2(ii, _):
            for u in range(U):
                sl = pl.ds((ii * U + u) * L, L)
                t1 = rstd * gb_v[0, sl]
                t2 = gb_v[1, sl] - mu * t1
                obuf[slot, j, sl] = obuf[slot, j, sl] * t1 + t2
            return 0
        lax.fori_loop(0, HV // U, p2, 0)

    def chunk(bw, bs, c, slot, j0):
        for t in range(CT):
            token(bw, bs, t, slot, j0 + t)

    def body(q, _):
        c0 = 4 * q

        @pl.when(q >= 1)
        def _():
            out_desc(2 * q - 2, 0, s_o0).wait()
        gather_wait(c0, bw_a, bs_a, s_a)
        chunk(bw_a, bs_a, c0, 0, 0)
        gather(c0 + 2, bw_a, bs_a, s_a)
        gather_wait(c0 + 1, bw_b, bs_b, s_b)
        chunk(bw_b, bs_b, c0 + 1, 0, 4)
        gather(c0 + 3, bw_b, bs_b, s_b)
        out_desc(2 * q, 0, s_o0).start()

        @pl.when(q >= 1)
        def _():
            out_desc(2 * q - 1, 1, s_o1).wait()
        gather_wait(c0 + 2, bw_a, bs_a, s_a)
        chunk(bw_a, bs_a, c0 + 2, 1, 0)

        @pl.when(q < NQ - 1)
        def _():
            gather(c0 + 4, bw_a, bs_a, s_a)
        gather_wait(c0 + 3, bw_b, bs_b, s_b)
        chunk(bw_b, bs_b, c0 + 3, 1, 4)

        @pl.when(q < NQ - 1)
        def _():
            gather(c0 + 5, bw_b, bs_b, s_b)
        out_desc(2 * q + 1, 1, s_o1).start()
        return 0

    gather(0, bw_a, bs_a, s_a)
    gather(1, bw_b, bs_b, s_b)
    lax.fori_loop(0, NQ, body, 0)
    out_desc(2 * NQ - 2, 0, s_o0).wait()
    out_desc(2 * NQ - 1, 1, s_o1).wait()


@functools.cache
def _build():
    mesh = plsc.VectorSubcoreMesh(core_axis_name="c", subcore_axis_name="s",
                                  num_cores=NC, num_subcores=NS)
    return pl.kernel(
        _body,
        out_type=jax.ShapeDtypeStruct((N, H), jnp.float32),
        mesh=mesh,
        scratch_types=[
            pltpu.VMEM((NCH * GS,), jnp.int32),
            pltpu.VMEM((8, H), jnp.float32),
            pltpu.VMEM((40, H), jnp.float32),
            pltpu.VMEM((8, H), jnp.float32),
            pltpu.VMEM((40, H), jnp.float32),
            pltpu.VMEM((2, 8, H), jnp.float32),
            pltpu.VMEM((2, H), jnp.float32),
            pltpu.SemaphoreType.DMA,
            pltpu.SemaphoreType.DMA,
            pltpu.SemaphoreType.DMA,
            pltpu.SemaphoreType.DMA,
        ],
    )


def kernel(input_ids, bbox, token_type_ids, position_ids, font_ids,
           word_emb, pos_emb, x_emb, y_emb, h_emb, w_emb, tok_emb, font_emb,
           gamma, beta):
    B, S = input_ids.shape
    i32 = jnp.int32
    small = jnp.concatenate([pos_emb, tok_emb, font_emb,
                             x_emb, y_emb, h_emb, w_emb], axis=0)
    ids = input_ids.reshape(N).astype(i32)
    pos_idx = jnp.broadcast_to(position_ids, (B, S)).reshape(N).astype(i32)
    tok_idx = token_type_ids.reshape(N).astype(i32)
    font_idx = font_ids.reshape(N).astype(i32)
    bb = bbox.astype(i32)
    left = bb[:, :, 0].reshape(N)
    upper = bb[:, :, 1].reshape(N)
    right = bb[:, :, 2].reshape(N)
    lower = bb[:, :, 3].reshape(N)
    idx9 = jnp.stack([
        pos_idx + _OFF[0],
        tok_idx + _OFF[1],
        font_idx + _OFF[2],
        left + _OFF[3],
        upper + _OFF[4],
        right + _OFF[3],
        lower + _OFF[4],
        (lower - upper) + _OFF[5],
        (right - left) + _OFF[6],
    ])
    idw = ids.reshape(NW, NCH, 1, CT)
    ism = idx9.reshape(9, NW, NCH, CT).transpose(1, 2, 0, 3)
    pad = jnp.zeros((NW, NCH, 1, CT), i32)
    idx = jnp.concatenate(
        [idw, pad, ism, pad], axis=2).reshape(-1)
    gb = jnp.stack([gamma, beta])
    out = _build()(word_emb, small, idx, gb)
    return out.reshape(B, S, H)


# speedup vs baseline: 1.0062x; 1.0062x over previous
import functools

import jax
import jax.numpy as jnp
from jax import lax
from jax.experimental import pallas as pl
from jax.experimental.pallas import tpu as pltpu
from jax.experimental.pallas import tpu_sc as plsc

N = 32768
H = 768
L = 16
HV = H // L
NC, NS = 2, 16
NW = NC * NS
NPW = N // NW
CT = 4
G = 10
GR = G * CT
NCH = NPW // CT
NQ = NCH // 4
INV_H = 1.0 / H
EPS = 1e-12

_SIZES = (512, 2, 128, 1024, 1024, 1024, 1024)
_OFF = []
_acc = 0
for _s in _SIZES:
    _OFF.append(_acc)
    _acc += _s
R_SMALL = _acc
GS = 48


def _rsqrt(x):
    xi = lax.bitcast_convert_type(x, jnp.int32)
    y = lax.bitcast_convert_type(0x5F3759DF - (xi >> 1), jnp.float32)
    for _ in range(3):
        y = y * (1.5 - 0.5 * x * y * y)
    return y


def _reduce_splat(v):
    dnums = lax.GatherDimensionNumbers(
        offset_dims=(), collapsed_slice_dims=(0,), start_index_map=(0,))
    for off in (8, 4, 2, 1):
        perm = lax.iota(jnp.int32, L) ^ off
        v = v + lax.gather(v, perm[:, None], dnums, (1,),
                           mode=lax.GatherScatterMode.PROMISE_IN_BOUNDS)
    return v


def _body(word_hbm, small_hbm, idx_hbm, gb_hbm, out_hbm,
          idx_v, bw_a, bs_a, bw_b, bs_b, obuf, gb_v, s_a, s_b, s_o0, s_o1):
    wid = lax.axis_index("s") * NC + lax.axis_index("c")
    base = wid * NPW
    pltpu.sync_copy(gb_hbm, gb_v)
    pltpu.sync_copy(idx_hbm.at[pl.ds(wid * (NCH * GS), NCH * GS)], idx_v)

    def _descs(c, bw, bs, sem):
        word = pltpu.make_async_copy(
            word_hbm.at[idx_v.at[pl.ds(c * GS, 8)]], bw, sem)
        small = pltpu.make_async_copy(
            small_hbm.at[idx_v.at[pl.ds(c * GS + 8, 40)]], bs, sem)
        return word, small

    def gather(c, bw, bs, sem):
        for d in _descs(c, bw, bs, sem):
            d.start()

    def gather_wait(c, bw, bs, sem):
        for d in _descs(c, bw, bs, sem):
            d.wait()

    def out_desc(p, slot, sem):
        return pltpu.make_async_copy(
            obuf.at[slot], out_hbm.at[pl.ds(base + p * 8, 8)], sem)

    U = 1

    def token(bw, bs, t, slot, j):
        refs = [(bw, t)] + [(bs, k * CT + t) for k in range(G - 1)]

        def p1(ii, carry):
            s, q = carry
            for u in range(U):
                sl = pl.ds((ii * U + u) * L, L)
                vs = [ref[r, sl] for ref, r in refs]
                while len(vs) > 1:
                    vs = [a + b for a, b in zip(vs[::2], vs[1::2])] + (
                        [vs[-1]] if len(vs) % 2 else [])
                v = vs[0]
                obuf[slot, j, sl] = v
                s = s + v
                q = q + v * v
            return s, q
        z = jnp.zeros((L,), jnp.float32)
        s, q = lax.fori_loop(0, HV // U, p1, (z, z))
        mu = _reduce_splat(s) * INV_H
        var = _reduce_splat(q) * INV_H - mu * mu
        rstd = _rsqrt(var + EPS)

        def p2(ii, _):
            for u in range(U):
                sl = pl.ds((ii * U + u) * L, L)
                t1 = rstd * gb_v[0, sl]
                t2 = gb_v[1, sl] - mu * t1
                obuf[slot, j, sl] = obuf[slot, j, sl] * t1 + t2
            return 0
        lax.fori_loop(0, HV // U, p2, 0)

    def chunk(bw, bs, c, slot, j0):
        for t in range(CT):
            token(bw, bs, t, slot, j0 + t)

    def body(q, _):
        c0 = 4 * q

        @pl.when(q >= 1)
        def _():
            out_desc(2 * q - 2, 0, s_o0).wait()
        gather_wait(c0, bw_a, bs_a, s_a)
        chunk(bw_a, bs_a, c0, 0, 0)
        gather(c0 + 2, bw_a, bs_a, s_a)
        gather_wait(c0 + 1, bw_b, bs_b, s_b)
        chunk(bw_b, bs_b, c0 + 1, 0, 4)
        gather(c0 + 3, bw_b, bs_b, s_b)
        out_desc(2 * q, 0, s_o0).start()

        @pl.when(q >= 1)
        def _():
            out_desc(2 * q - 1, 1, s_o1).wait()
        gather_wait(c0 + 2, bw_a, bs_a, s_a)
        chunk(bw_a, bs_a, c0 + 2, 1, 0)

        @pl.when(q < NQ - 1)
        def _():
            gather(c0 + 4, bw_a, bs_a, s_a)
        gather_wait(c0 + 3, bw_b, bs_b, s_b)
        chunk(bw_b, bs_b, c0 + 3, 1, 4)

        @pl.when(q < NQ - 1)
        def _():
            gather(c0 + 5, bw_b, bs_b, s_b)
        out_desc(2 * q + 1, 1, s_o1).start()
        return 0

    gather(0, bw_a, bs_a, s_a)
    gather(1, bw_b, bs_b, s_b)
    lax.fori_loop(0, NQ, body, 0)
    out_desc(2 * NQ - 2, 0, s_o0).wait()
    out_desc(2 * NQ - 1, 1, s_o1).wait()


@functools.cache
def _build():
    mesh = plsc.VectorSubcoreMesh(core_axis_name="c", subcore_axis_name="s",
                                  num_cores=NC, num_subcores=NS)
    return pl.kernel(
        _body,
        out_type=jax.ShapeDtypeStruct((N, H), jnp.float32),
        mesh=mesh,
        scratch_types=[
            pltpu.VMEM((NCH * GS,), jnp.int32),
            pltpu.VMEM((8, H), jnp.float32),
            pltpu.VMEM((40, H), jnp.float32),
            pltpu.VMEM((8, H), jnp.float32),
            pltpu.VMEM((40, H), jnp.float32),
            pltpu.VMEM((2, 8, H), jnp.float32),
            pltpu.VMEM((2, H), jnp.float32),
            pltpu.SemaphoreType.DMA,
            pltpu.SemaphoreType.DMA,
            pltpu.SemaphoreType.DMA,
            pltpu.SemaphoreType.DMA,
        ],
    )


def kernel(input_ids, bbox, token_type_ids, position_ids, font_ids,
           word_emb, pos_emb, x_emb, y_emb, h_emb, w_emb, tok_emb, font_emb,
           gamma, beta):
    B, S = input_ids.shape
    i32 = jnp.int32
    small = jnp.concatenate([pos_emb, tok_emb, font_emb,
                             x_emb, y_emb, h_emb, w_emb], axis=0)
    ids = input_ids.reshape(N).astype(i32)
    pos_idx = jnp.broadcast_to(position_ids, (B, S)).reshape(N).astype(i32)
    tok_idx = token_type_ids.reshape(N).astype(i32)
    font_idx = font_ids.reshape(N).astype(i32)
    bb = bbox.astype(i32)
    left = bb[:, :, 0].reshape(N)
    upper = bb[:, :, 1].reshape(N)
    right = bb[:, :, 2].reshape(N)
    lower = bb[:, :, 3].reshape(N)
    idx9 = jnp.stack([
        pos_idx + _OFF[0],
        tok_idx + _OFF[1],
        font_idx + _OFF[2],
        left + _OFF[3],
        upper + _OFF[4],
        right + _OFF[3],
        lower + _OFF[4],
        (lower - upper) + _OFF[5],
        (right - left) + _OFF[6],
    ])
    idw = ids.reshape(NW, NCH, 1, CT)
    ism = idx9.reshape(9, NW, NCH, CT).transpose(1, 2, 0, 3)
    pad = jnp.zeros((NW, NCH, 1, CT), i32)
    idx = jnp.concatenate(
        [idw, pad, ism, pad], axis=2).reshape(-1)
    gb = jnp.stack([gamma, beta])
    out = _build()(word_emb, small, idx, gb)
    return out.reshape(B, S, H)


# speedup vs baseline: 1.7311x; 1.7204x over previous
import functools

import jax
import jax.numpy as jnp
from jax import lax
from jax.experimental import pallas as pl
from jax.experimental.pallas import tpu as pltpu
from jax.experimental.pallas import tpu_sc as plsc

N = 32768
H = 768
L = 16
HV = H // L
NC, NS = 2, 16
NW = NC * NS
NPW = N // NW
CT = 8
G = 9
GS = G * CT
NCH = NPW // CT
NQ = NCH // 2
INV_H = 1.0 / H
EPS = 1e-12

_SIZES = (1024, 128, 1024, 1024, 1024, 1024)
_OFF = []
_acc = 0
for _s in _SIZES:
    _OFF.append(_acc)
    _acc += _s


def _rsqrt(x):
    xi = lax.bitcast_convert_type(x, jnp.int32)
    y = lax.bitcast_convert_type(0x5F3759DF - (xi >> 1), jnp.float32)
    for _ in range(3):
        y = y * (1.5 - 0.5 * x * y * y)
    return y


def _reduce_splat(v):
    dnums = lax.GatherDimensionNumbers(
        offset_dims=(), collapsed_slice_dims=(0,), start_index_map=(0,))
    for off in (8, 4, 2, 1):
        perm = lax.iota(jnp.int32, L) ^ off
        v = v + lax.gather(v, perm[:, None], dnums, (1,),
                           mode=lax.GatherScatterMode.PROMISE_IN_BOUNDS)
    return v


def _body(word_hbm, small_hbm, idx_hbm, gb_hbm, out_hbm,
          idx_v, buf_a, buf_b, obuf, gb_v, s_a, s_b, s_o):
    wid = lax.axis_index("s") * NC + lax.axis_index("c")
    base = wid * NPW
    pltpu.sync_copy(gb_hbm, gb_v)
    pltpu.sync_copy(idx_hbm.at[pl.ds(wid * (NCH * GS), NCH * GS)], idx_v)

    def _descs(c, buf, sem):
        small = pltpu.make_async_copy(
            small_hbm.at[idx_v.at[pl.ds(c * GS + CT, (G - 1) * CT)]],
            buf.at[pl.ds(0, (G - 1) * CT)], sem)
        word = pltpu.make_async_copy(
            word_hbm.at[idx_v.at[pl.ds(c * GS, CT)]],
            buf.at[pl.ds((G - 1) * CT, CT)], sem)
        return small, word

    def gather(c, buf, sem):
        for d in _descs(c, buf, sem):
            d.start()

    def gather_wait(c, buf, sem):
        for d in _descs(c, buf, sem):
            d.wait()

    def out_desc(c):
        return pltpu.make_async_copy(
            obuf, out_hbm.at[pl.ds(base + c * CT, CT)], s_o)

    U = 2

    def token(buf, t):
        rows = [(G - 1) * CT + t] + [k * CT + t for k in range(G - 1)]

        def p1(ii, carry):
            s, q = carry
            for u in range(U):
                sl = pl.ds((ii * U + u) * L, L)
                vs = [buf[r, sl] for r in rows]
                while len(vs) > 1:
                    vs = [a + b for a, b in zip(vs[::2], vs[1::2])] + (
                        [vs[-1]] if len(vs) % 2 else [])
                v = vs[0]
                obuf[t, sl] = v
                s = s + v
                q = q + v * v
            return s, q
        z = jnp.zeros((L,), jnp.float32)
        s, q = lax.fori_loop(0, HV // U, p1, (z, z))
        mu = _reduce_splat(s) * INV_H
        var = _reduce_splat(q) * INV_H - mu * mu
        rstd = _rsqrt(var + EPS)

        def p2(ii, _):
            for u in range(U):
                sl = pl.ds((ii * U + u) * L, L)
                t1 = rstd * gb_v[0, sl]
                t2 = gb_v[1, sl] - mu * t1
                obuf[t, sl] = obuf[t, sl] * t1 + t2
            return 0
        lax.fori_loop(0, HV // U, p2, 0)

    def chunk(buf):
        for t in range(CT):
            token(buf, t)

    def body(q, _):
        c0 = 2 * q
        gather_wait(c0, buf_a, s_a)

        @pl.when(q >= 1)
        def _():
            out_desc(c0 - 1).wait()
        chunk(buf_a)
        out_desc(c0).start()

        @pl.when(q < NQ - 1)
        def _():
            gather(c0 + 2, buf_a, s_a)
        gather_wait(c0 + 1, buf_b, s_b)
        out_desc(c0).wait()
        chunk(buf_b)
        out_desc(c0 + 1).start()

        @pl.when(q < NQ - 1)
        def _():
            gather(c0 + 3, buf_b, s_b)
        return 0

    gather(0, buf_a, s_a)
    gather(1, buf_b, s_b)
    lax.fori_loop(0, NQ, body, 0)
    out_desc(NCH - 1).wait()


@functools.cache
def _build():
    mesh = plsc.VectorSubcoreMesh(core_axis_name="c", subcore_axis_name="s",
                                  num_cores=NC, num_subcores=NS)
    return pl.kernel(
        _body,
        out_type=jax.ShapeDtypeStruct((N, H), jnp.float32),
        mesh=mesh,
        scratch_types=[
            pltpu.VMEM((NCH * GS,), jnp.int32),
            pltpu.VMEM((GS, H), jnp.float32),
            pltpu.VMEM((GS, H), jnp.float32),
            pltpu.VMEM((CT, H), jnp.float32),
            pltpu.VMEM((2, H), jnp.float32),
            pltpu.SemaphoreType.DMA,
            pltpu.SemaphoreType.DMA,
            pltpu.SemaphoreType.DMA,
        ],
    )


def kernel(input_ids, bbox, token_type_ids, position_ids, font_ids,
           word_emb, pos_emb, x_emb, y_emb, h_emb, w_emb, tok_emb, font_emb,
           gamma, beta):
    B, S = input_ids.shape
    i32 = jnp.int32
    postok = (tok_emb[:, None, :] + pos_emb[None, :, :]).reshape(-1, H)
    small = jnp.concatenate([postok, font_emb, x_emb, y_emb, h_emb, w_emb],
                            axis=0)
    ids = input_ids.reshape(N).astype(i32)
    pos_idx = jnp.broadcast_to(position_ids, (B, S)).reshape(N).astype(i32)
    tok_idx = token_type_ids.reshape(N).astype(i32)
    font_idx = font_ids.reshape(N).astype(i32)
    bb = bbox.astype(i32)
    left = bb[:, :, 0].reshape(N)
    upper = bb[:, :, 1].reshape(N)
    right = bb[:, :, 2].reshape(N)
    lower = bb[:, :, 3].reshape(N)
    idx8 = jnp.stack([
        tok_idx * 512 + pos_idx + _OFF[0],
        font_idx + _OFF[1],
        left + _OFF[2],
        upper + _OFF[3],
        right + _OFF[2],
        lower + _OFF[3],
        (lower - upper) + _OFF[4],
        (right - left) + _OFF[5],
    ])
    idw = ids.reshape(NW, NCH, 1, CT)
    ism = idx8.reshape(8, NW, NCH, CT).transpose(1, 2, 0, 3)
    idx = jnp.concatenate([idw, ism], axis=2).reshape(-1)
    gb = jnp.stack([gamma, beta])
    out = _build()(word_emb, small, idx, gb)
    return out.reshape(B, S, H)
